# baseline (speedup 1.0000x reference)
import functools

import jax
import jax.numpy as jnp
from jax import lax
from jax.experimental import pallas as pl
from jax.experimental.pallas import tpu as pltpu
from jax.experimental.pallas import tpu_sc as plsc

_B = 16
_T = 2048
_L = 16
_CHUNKS = _T // _L

_DNUMS = lax.GatherDimensionNumbers(
    offset_dims=(), collapsed_slice_dims=(0,), start_index_map=(0,))


def _permute(x, idx):
    return lax.gather(x, idx[:, None], dimension_numbers=_DNUMS,
                      slice_sizes=(1,),
                      mode=lax.GatherScatterMode.PROMISE_IN_BOUNDS)


def _jitter_body(pitch_hbm, out_hbm, row_v, out_v, sem):
    s = lax.axis_index("s")
    half = _T // 2
    cp2 = pltpu.async_copy(
        pitch_hbm.at[s, pl.ds(half, half)], row_v.at[pl.ds(half, half)], sem)
    pltpu.sync_copy(pitch_hbm.at[s, pl.ds(0, half)], row_v.at[pl.ds(0, half)])
    iota = lax.iota(jnp.int32, _L)
    zeros_i = jnp.zeros((_L,), jnp.int32)

    def step(i, carry):
        cnt, acc, dacc, prev = carry
        f0 = row_v[pl.ds(i * _L, _L)]
        m = f0 > 0.0
        t0 = 1.0 / (f0 + 1e-5)
        acc = acc + jnp.where(m, t0, 0.0)
        keys = jnp.where(m, iota, _L + iota)
        _, sv = plsc.sort_key_val(keys, t0)
        pop = plsc.all_reduce_population_count(m)
        sh = _permute(sv, jnp.maximum(iota - 1, 0))
        vprev = jnp.where(iota == 0, prev, sh)
        dmask = (iota < pop) & ((iota > 0) | (cnt > 0))
        dacc = dacc + jnp.where(dmask, jnp.abs(sv - vprev), 0.0)
        lastv = _permute(sv, jnp.maximum(pop - 1, 0))
        prev = jnp.where(pop > 0, lastv, prev)
        return cnt + pop, acc, dacc, prev

    carry = lax.fori_loop(
        0, _CHUNKS // 2, step,
        (zeros_i, jnp.zeros((_L,), jnp.float32),
         jnp.zeros((_L,), jnp.float32), jnp.zeros((_L,), jnp.float32)),
        unroll=2)
    cp2.wait()
    cnt, acc, dacc, _ = lax.fori_loop(
        _CHUNKS // 2, _CHUNKS, step, carry, unroll=2)

    def tree_sum(x):
        for d in (1, 2, 4, 8):
            x = x + _permute(x, (iota + d) & (_L - 1))
        return x

    sum_valid = tree_sum(acc)
    sum_diffs = tree_sum(dacc)
    cntf = cnt.astype(jnp.float32)
    mean_t0 = sum_valid / jnp.maximum(cntf, 1.0)
    mean_d = sum_diffs / jnp.maximum(cntf - 1.0, 1.0)
    jit = jnp.where(cnt >= 2, mean_d / (mean_t0 + 1e-8), 0.0)
    out_v[...] = jnp.where(iota == 0, jit, 0.0)
    pltpu.sync_copy(out_v, out_hbm.at[s])


_jitter_call = pl.kernel(
    _jitter_body,
    out_type=jax.ShapeDtypeStruct((_B, _L), jnp.float32),
    mesh=plsc.VectorSubcoreMesh(
        core_axis_name="c", subcore_axis_name="s", num_cores=1),
    scratch_types=[
        pltpu.VMEM((_T,), jnp.float32),
        pltpu.VMEM((_L,), jnp.float32),
        pltpu.SemaphoreType.DMA,
    ],
    compiler_params=pltpu.CompilerParams(needs_layout_passes=False),
)


def kernel(waveform, pitch_f0):
    del waveform
    out16 = _jitter_call(pitch_f0)
    return out16[:, :3]


# speedup vs baseline: 1.0050x; 1.0050x over previous
import jax
import jax.numpy as jnp
from jax import lax
from jax.experimental import pallas as pl
from jax.experimental.pallas import tpu as pltpu
from jax.experimental.pallas import tpu_sc as plsc

_B = 16
_T = 2048
_L = 16
_CHUNKS = _T // _L

_DNUMS = lax.GatherDimensionNumbers(
    offset_dims=(), collapsed_slice_dims=(0,), start_index_map=(0,))


def _permute(x, idx):
    return lax.gather(x, idx[:, None], dimension_numbers=_DNUMS,
                      slice_sizes=(1,),
                      mode=lax.GatherScatterMode.PROMISE_IN_BOUNDS)


def _jitter_body(pitch_hbm, out_hbm, row_v, out_v):
    s = lax.axis_index("s")
    pltpu.sync_copy(pitch_hbm.at[s], row_v)
    iota = lax.iota(jnp.int32, _L)
    zeros_i = jnp.zeros((_L,), jnp.int32)

    def step(i, carry):
        cnt, acc, dacc, prev = carry
        f0 = row_v[pl.ds(i * _L, _L)]
        m = f0 > 0.0
        t0 = 1.0 / (f0 + 1e-5)
        acc = acc + jnp.where(m, t0, 0.0)
        keys = jnp.where(m, iota, _L + iota)
        _, sv = plsc.sort_key_val(keys, t0)
        pop = plsc.all_reduce_population_count(m)
        sh = _permute(sv, jnp.maximum(iota - 1, 0))
        vprev = jnp.where(iota == 0, prev, sh)
        dmask = (iota < pop) & ((iota > 0) | (cnt > 0))
        dacc = dacc + jnp.where(dmask, jnp.abs(sv - vprev), 0.0)
        lastv = _permute(sv, jnp.maximum(pop - 1, 0))
        prev = jnp.where(pop > 0, lastv, prev)
        return cnt + pop, acc, dacc, prev

    cnt, acc, dacc, _ = lax.fori_loop(
        0, _CHUNKS, step,
        (zeros_i, jnp.zeros((_L,), jnp.float32),
         jnp.zeros((_L,), jnp.float32), jnp.zeros((_L,), jnp.float32)),
        unroll=2)

    def tree_sum(x):
        for d in (1, 2, 4, 8):
            x = x + _permute(x, (iota + d) & (_L - 1))
        return x

    sum_valid = tree_sum(acc)
    sum_diffs = tree_sum(dacc)
    cntf = cnt.astype(jnp.float32)
    mean_t0 = sum_valid / jnp.maximum(cntf, 1.0)
    mean_d = sum_diffs / jnp.maximum(cntf - 1.0, 1.0)
    jit = jnp.where(cnt >= 2, mean_d / (mean_t0 + 1e-8), 0.0)
    out_v[...] = jnp.where(iota == 0, jit, 0.0)
    pltpu.sync_copy(out_v, out_hbm.at[s])


_jitter_call = pl.kernel(
    _jitter_body,
    out_type=jax.ShapeDtypeStruct((_B, _L), jnp.float32),
    mesh=plsc.VectorSubcoreMesh(
        core_axis_name="c", subcore_axis_name="s", num_cores=1),
    scratch_types=[
        pltpu.VMEM((_T,), jnp.float32),
        pltpu.VMEM((_L,), jnp.float32),
    ],
    compiler_params=pltpu.CompilerParams(needs_layout_passes=False),
)


def kernel(waveform, pitch_f0):
    del waveform
    out16 = _jitter_call(pitch_f0)
    return out16[:, :3]
